# baseline (speedup 1.0000x reference)
import jax
import jax.numpy as jnp
from jax import lax
from jax.experimental import pallas as pl
from jax.experimental.pallas import tpu as pltpu, tpu_sc as plsc

_NUM_CLASSES = 100000
_DIM = 64
_BATCH = 16384

_INFO = plsc.get_sparse_core_info()
_NC = _INFO.num_cores
_NS = _INFO.num_subcores
_NW = _NC * _NS
_CPT = _DIM // _NW
_LANES = 16
_FCH = 4096
_NCH = _BATCH // _FCH * _CPT
_UNROLL = 16


def _body(ct_hbm, y_hbm, ft_hbm, out_hbm, col_v, idx_v, f_v, acc_v,
          csem, isem, fsem):
    c = lax.axis_index("c")
    s = lax.axis_index("s")
    wid = s * _NC + c
    col0 = wid * _CPT
    half = _NCH // _CPT

    def col_copy(ci):
        return pltpu.make_async_copy(ct_hbm.at[col0 + ci], col_v, csem)

    def f_copy(t, buf):
        col = col0 + lax.div(t, half)
        ch = lax.rem(t, half)
        return pltpu.make_async_copy(
            ft_hbm.at[col, pl.ds(ch * _FCH, _FCH)],
            f_v.at[pl.ds(buf * _FCH, _FCH)], fsem.at[buf])

    pltpu.make_async_copy(y_hbm, idx_v, isem).start()
    col_copy(0).start()
    f_copy(0, 0).start()
    pltpu.make_async_copy(y_hbm, idx_v, isem).wait()

    accs = (jnp.zeros((_LANES,), jnp.float32),) * _UNROLL

    def chunk(t, accs):
        buf = lax.rem(t, 2)

        @pl.when(t == 0)
        def _():
            col_copy(0).wait()

        @pl.when(t == half)
        def _():
            col_copy(1).start()
            col_copy(1).wait()

        @pl.when(t + 1 < _NCH)
        def _():
            f_copy(t + 1, 1 - buf).start()

        f_copy(t, buf).wait()
        ibase = lax.rem(t, half) * _FCH
        fbase = buf * _FCH

        def step(off, accs):
            new = []
            for u in range(_UNROLL):
                o = off + u * _LANES
                g = plsc.load_gather(col_v, [idx_v[pl.ds(ibase + o, _LANES)]])
                d = f_v[pl.ds(fbase + o, _LANES)] - g
                new.append(accs[u] + d * d)
            return tuple(new)

        return plsc.parallel_loop(
            0, _FCH, step=_UNROLL * _LANES, carry=accs)(step)

    accs = lax.fori_loop(0, _NCH, chunk, accs)

    total = accs[0]
    for u in range(1, _UNROLL):
        total = total + accs[u]
    acc_v[...] = total
    pltpu.sync_copy(acc_v, out_hbm.at[wid])


_sc_call = pl.kernel(
    _body,
    out_type=jax.ShapeDtypeStruct((_NW, _LANES), jnp.float32),
    mesh=plsc.VectorSubcoreMesh(core_axis_name="c", subcore_axis_name="s"),
    compiler_params=pltpu.CompilerParams(
        needs_layout_passes=False, skip_device_barrier=True),
    scratch_types=[
        pltpu.VMEM((_NUM_CLASSES,), jnp.float32),
        pltpu.VMEM((_BATCH,), jnp.int32),
        pltpu.VMEM((2 * _FCH,), jnp.float32),
        pltpu.VMEM((_LANES,), jnp.float32),
        pltpu.SemaphoreType.DMA,
        pltpu.SemaphoreType.DMA,
        pltpu.SemaphoreType.DMA((2,)),
    ],
)


@jax.jit
def kernel(f, y, centers):
    partials = _sc_call(centers.T, y.astype(jnp.int32), f.T)
    return jnp.sum(partials) * (1.0 / _BATCH)


# speedup vs baseline: 1.0088x; 1.0088x over previous
import jax
import jax.numpy as jnp
from jax import lax
from jax.experimental import pallas as pl
from jax.experimental.pallas import tpu as pltpu, tpu_sc as plsc

_NUM_CLASSES = 100000
_DIM = 64
_BATCH = 16384

_INFO = plsc.get_sparse_core_info()
_NC = _INFO.num_cores
_NS = _INFO.num_subcores
_NW = _NC * _NS
_CPT = _DIM // _NW
_LANES = 16
_FCH = 4096
_NCH = _BATCH // _FCH * _CPT
_UNROLL = 4


def _body(ct_hbm, y_hbm, ft_hbm, out_hbm, col_v, idx_v, f_v, acc_v,
          csem, isem, fsem):
    c = lax.axis_index("c")
    s = lax.axis_index("s")
    wid = s * _NC + c
    col0 = wid * _CPT
    half = _NCH // _CPT

    def col_copy(ci):
        return pltpu.make_async_copy(ct_hbm.at[col0 + ci], col_v, csem)

    def f_copy(t, buf):
        col = col0 + lax.div(t, half)
        ch = lax.rem(t, half)
        return pltpu.make_async_copy(
            ft_hbm.at[col, pl.ds(ch * _FCH, _FCH)],
            f_v.at[pl.ds(buf * _FCH, _FCH)], fsem.at[buf])

    pltpu.make_async_copy(y_hbm, idx_v, isem).start()
    col_copy(0).start()
    f_copy(0, 0).start()
    pltpu.make_async_copy(y_hbm, idx_v, isem).wait()

    accs = (jnp.zeros((_LANES,), jnp.float32),) * _UNROLL

    def chunk(t, accs):
        buf = lax.rem(t, 2)

        @pl.when(t == 0)
        def _():
            col_copy(0).wait()

        @pl.when(t == half)
        def _():
            col_copy(1).start()
            col_copy(1).wait()

        @pl.when(t + 1 < _NCH)
        def _():
            f_copy(t + 1, 1 - buf).start()

        f_copy(t, buf).wait()
        ibase = lax.rem(t, half) * _FCH
        fbase = buf * _FCH

        def step(off, accs):
            new = []
            for u in range(_UNROLL):
                o = off + u * _LANES
                g = plsc.load_gather(col_v, [idx_v[pl.ds(ibase + o, _LANES)]])
                d = f_v[pl.ds(fbase + o, _LANES)] - g
                new.append(accs[u] + d * d)
            return tuple(new)

        return plsc.parallel_loop(
            0, _FCH, step=_UNROLL * _LANES, carry=accs)(step)

    accs = lax.fori_loop(0, _NCH, chunk, accs)

    total = accs[0]
    for u in range(1, _UNROLL):
        total = total + accs[u]
    acc_v[...] = total
    pltpu.sync_copy(acc_v, out_hbm.at[wid])


_sc_call = pl.kernel(
    _body,
    out_type=jax.ShapeDtypeStruct((_NW, _LANES), jnp.float32),
    mesh=plsc.VectorSubcoreMesh(core_axis_name="c", subcore_axis_name="s"),
    compiler_params=pltpu.CompilerParams(
        needs_layout_passes=False, skip_device_barrier=True),
    scratch_types=[
        pltpu.VMEM((_NUM_CLASSES,), jnp.float32),
        pltpu.VMEM((_BATCH,), jnp.int32),
        pltpu.VMEM((2 * _FCH,), jnp.float32),
        pltpu.VMEM((_LANES,), jnp.float32),
        pltpu.SemaphoreType.DMA,
        pltpu.SemaphoreType.DMA,
        pltpu.SemaphoreType.DMA((2,)),
    ],
)


@jax.jit
def kernel(f, y, centers):
    partials = _sc_call(centers.T, y.astype(jnp.int32), f.T)
    return jnp.sum(partials) * (1.0 / _BATCH)


# speedup vs baseline: 1.0100x; 1.0012x over previous
import jax
import jax.numpy as jnp
from jax import lax
from jax.experimental import pallas as pl
from jax.experimental.pallas import tpu as pltpu, tpu_sc as plsc

_NUM_CLASSES = 100000
_DIM = 64
_BATCH = 16384

_INFO = plsc.get_sparse_core_info()
_NC = _INFO.num_cores
_NS = _INFO.num_subcores
_NW = _NC * _NS
_CPT = _DIM // _NW
_LANES = 16
_FCH = 4096
_NCH = _BATCH // _FCH * _CPT
_UNROLL = 8


def _body(ct_hbm, y_hbm, ft_hbm, out_hbm, col_v, idx_v, f_v, acc_v,
          csem, isem, fsem):
    c = lax.axis_index("c")
    s = lax.axis_index("s")
    wid = s * _NC + c
    col0 = wid * _CPT
    half = _NCH // _CPT

    def col_copy(ci):
        return pltpu.make_async_copy(ct_hbm.at[col0 + ci], col_v, csem)

    def f_copy(t, buf):
        col = col0 + lax.div(t, half)
        ch = lax.rem(t, half)
        return pltpu.make_async_copy(
            ft_hbm.at[col, pl.ds(ch * _FCH, _FCH)],
            f_v.at[pl.ds(buf * _FCH, _FCH)], fsem.at[buf])

    pltpu.make_async_copy(y_hbm, idx_v, isem).start()
    col_copy(0).start()
    f_copy(0, 0).start()
    pltpu.make_async_copy(y_hbm, idx_v, isem).wait()

    accs = (jnp.zeros((_LANES,), jnp.float32),) * _UNROLL

    def chunk(t, accs):
        buf = lax.rem(t, 2)

        @pl.when(t == 0)
        def _():
            col_copy(0).wait()

        @pl.when(t == half)
        def _():
            col_copy(1).start()
            col_copy(1).wait()

        @pl.when(t + 1 < _NCH)
        def _():
            f_copy(t + 1, 1 - buf).start()

        f_copy(t, buf).wait()
        ibase = lax.rem(t, half) * _FCH
        fbase = buf * _FCH

        def step(off, accs):
            new = []
            for u in range(_UNROLL):
                o = off + u * _LANES
                g = plsc.load_gather(col_v, [idx_v[pl.ds(ibase + o, _LANES)]])
                d = f_v[pl.ds(fbase + o, _LANES)] - g
                new.append(accs[u] + d * d)
            return tuple(new)

        return plsc.parallel_loop(
            0, _FCH, step=_UNROLL * _LANES, carry=accs)(step)

    accs = lax.fori_loop(0, _NCH, chunk, accs)

    total = accs[0]
    for u in range(1, _UNROLL):
        total = total + accs[u]
    acc_v[...] = total
    pltpu.sync_copy(acc_v, out_hbm.at[wid])


_sc_call = pl.kernel(
    _body,
    out_type=jax.ShapeDtypeStruct((_NW, _LANES), jnp.float32),
    mesh=plsc.VectorSubcoreMesh(core_axis_name="c", subcore_axis_name="s"),
    compiler_params=pltpu.CompilerParams(
        needs_layout_passes=False, skip_device_barrier=True),
    scratch_types=[
        pltpu.VMEM((_NUM_CLASSES,), jnp.float32),
        pltpu.VMEM((_BATCH,), jnp.int32),
        pltpu.VMEM((2 * _FCH,), jnp.float32),
        pltpu.VMEM((_LANES,), jnp.float32),
        pltpu.SemaphoreType.DMA,
        pltpu.SemaphoreType.DMA,
        pltpu.SemaphoreType.DMA((2,)),
    ],
)


@jax.jit
def kernel(f, y, centers):
    partials = _sc_call(centers.T, y.astype(jnp.int32), f.T)
    return jnp.sum(partials) * (1.0 / _BATCH)
